# baseline (speedup 1.0000x reference)
import jax
import jax.numpy as jnp
from jax.experimental import pallas as pl
from jax.experimental.pallas import tpu as pltpu

MAX_LEN = 2048
EMBED_DIM = 768
NCHUNK = 4
CHUNK = MAX_LEN // NCHUNK


def _copy_body(table_ref, out_ref, buf, *sems):
    gathers = []
    for i in range(NCHUNK):
        g = pltpu.make_async_copy(
            table_ref.at[pl.ds(i * CHUNK, CHUNK)], buf.at[i], sems[i]
        )
        g.start()
        gathers.append(g)
    puts = []
    for i in range(NCHUNK):
        gathers[i].wait()
        p = pltpu.make_async_copy(
            buf.at[i], out_ref.at[pl.ds(i * CHUNK, CHUNK)], sems[NCHUNK + i]
        )
        p.start()
        puts.append(p)
    for p in puts:
        p.wait()


@jax.jit
def _tc_copy(table):
    return pl.pallas_call(
        _copy_body,
        in_specs=[pl.BlockSpec(memory_space=pl.ANY)],
        out_specs=pl.BlockSpec(memory_space=pl.ANY),
        scratch_shapes=(
            [pltpu.VMEM((NCHUNK, CHUNK, EMBED_DIM), jnp.float32)]
            + [pltpu.SemaphoreType.DMA] * (2 * NCHUNK)
        ),
        out_shape=jax.ShapeDtypeStruct((MAX_LEN, EMBED_DIM), jnp.float32),
    )(table)


def kernel(x, table):
    del x
    return _tc_copy(table)[None]
